# baseline (speedup 1.0000x reference)
import functools

import jax
import jax.numpy as jnp
from jax import lax
from jax.experimental import pallas as pl
from jax.experimental.pallas import tpu as pltpu
from jax.experimental.pallas import tpu_sc as plsc

D = 384
N_NODE = 128
N_DEPTH = 32
N_CTAB = N_NODE * N_DEPTH
N_TOK = 4 * 8192

NC = 2
NS = 16
L = 16
NW = NC * NS
TOK_W = N_TOK // NW
NPW = N_NODE // NS
CH = 128
NCH = TOK_W // CH
NV = D // L


def _sc_body(nid_hbm, did_hbm, ntab_hbm, dtab_hbm, out_hbm, ctab_hbm,
             ntab_v, dtab_v, bld_v, nidx_v, didx_v, cidx_v, rows_v,
             gsem, ssem):
    core = lax.axis_index("c")
    sid = lax.axis_index("s")
    wid = sid * NC + core
    base = wid * TOK_W

    pltpu.sync_copy(ntab_hbm.at[pl.ds(sid * NPW * D, NPW * D)], ntab_v)
    pltpu.sync_copy(dtab_hbm, dtab_v)
    pltpu.sync_copy(nid_hbm.at[pl.ds(base, TOK_W)], nidx_v)
    pltpu.sync_copy(did_hbm.at[pl.ds(base, TOK_W)], didx_v)

    ctab_base = core * N_CTAB

    def _build(n, carry):
        nvs = [ntab_v[pl.ds(n * D + j * L, L)] for j in range(NV)]
        for d in range(N_DEPTH):
            for j in range(NV):
                s = pl.ds(j * L, L)
                bld_v[d, s] = nvs[j] + dtab_v[d, s]
        row0 = ctab_base + (sid * NPW + n) * N_DEPTH
        pltpu.sync_copy(bld_v, ctab_hbm.at[pl.ds(row0, N_DEPTH)])
        return carry

    lax.fori_loop(0, NPW, _build, 0)

    def _combine(i, carry):
        s = pl.ds(i * L, L)
        cidx_v[s] = nidx_v[s] * N_DEPTH + didx_v[s] + ctab_base
        return carry

    lax.fori_loop(0, TOK_W // L, _combine, 0)

    plsc.subcore_barrier()

    def _gather(c):
        idx = cidx_v.at[pl.ds(c * CH, CH)]
        return pltpu.async_copy(ctab_hbm.at[idx], rows_v.at[c % 2], gsem)

    def _scatter(c):
        return pltpu.async_copy(
            rows_v.at[c % 2], out_hbm.at[pl.ds(base + c * CH, CH)], ssem)

    gathers = [None] * NCH
    scatters = [None] * NCH
    gathers[0] = _gather(0)
    for c in range(NCH):
        gathers[c].wait()
        if c + 1 < NCH:
            if c - 1 >= 0:
                scatters[c - 1].wait()
            gathers[c + 1] = _gather(c + 1)
        scatters[c] = _scatter(c)
    scatters[NCH - 2].wait()
    scatters[NCH - 1].wait()


@jax.jit
def _run(node_ids, depth_ids, node_tab_flat, depth_tab):
    k = functools.partial(
        pl.kernel,
        out_type=(
            jax.ShapeDtypeStruct((N_TOK, D), jnp.float32),
            jax.ShapeDtypeStruct((NC * N_CTAB, D), jnp.float32),
        ),
        mesh=plsc.VectorSubcoreMesh(core_axis_name="c", subcore_axis_name="s"),
        scratch_types=[
            pltpu.VMEM((NPW * D,), jnp.float32),
            pltpu.VMEM((N_DEPTH, D), jnp.float32),
            pltpu.VMEM((N_DEPTH, D), jnp.float32),
            pltpu.VMEM((TOK_W,), jnp.int32),
            pltpu.VMEM((TOK_W,), jnp.int32),
            pltpu.VMEM((TOK_W,), jnp.int32),
            pltpu.VMEM((2, CH, D), jnp.float32),
            pltpu.SemaphoreType.DMA,
            pltpu.SemaphoreType.DMA,
        ],
    )(_sc_body)
    out, _ = k(node_ids, depth_ids, node_tab_flat, depth_tab)
    return out


def kernel(node_type_ids, depth_ids, node_table, depth_table):
    b, t = node_type_ids.shape
    nid = node_type_ids.reshape(-1).astype(jnp.int32)
    did = depth_ids.reshape(-1).astype(jnp.int32)
    out = _run(nid, did, node_table.reshape(-1), depth_table)
    return out.reshape(b, t, D)


# speedup vs baseline: 1.1577x; 1.1577x over previous
import functools

import jax
import jax.numpy as jnp
from jax import lax
from jax.experimental import pallas as pl
from jax.experimental.pallas import tpu as pltpu
from jax.experimental.pallas import tpu_sc as plsc

D = 384
N_NODE = 128
N_DEPTH = 32
N_TOK = 4 * 8192

NC = 2
NS = 16
L = 16
NW = NC * NS
TOK_W = N_TOK // NW
CH = 64
NCH = TOK_W // CH
NB = 4


def _ctable_body(node_ref, depth_ref, out_ref):
    node = node_ref[...]
    depth = depth_ref[...]
    out_ref[...] = node[:, None, :] + depth[None, :, :]


def _build_ctable(node_table, depth_table):
    out = pl.pallas_call(
        _ctable_body,
        out_shape=jax.ShapeDtypeStruct((N_NODE, N_DEPTH, D), jnp.float32),
    )(node_table, depth_table)
    return out.reshape(N_NODE * N_DEPTH, D)


def _sc_body(nid_hbm, did_hbm, ctab_hbm, out_hbm, nidx_v, didx_v, cidx_v,
             rows_v, gsem, ssem):
    wid = lax.axis_index("s") * NC + lax.axis_index("c")
    base = wid * TOK_W
    pltpu.sync_copy(nid_hbm.at[pl.ds(base, TOK_W)], nidx_v)
    pltpu.sync_copy(did_hbm.at[pl.ds(base, TOK_W)], didx_v)

    def _combine(i, carry):
        s = pl.ds(i * L, L)
        cidx_v[s] = nidx_v[s] * N_DEPTH + didx_v[s]
        return carry

    lax.fori_loop(0, TOK_W // L, _combine, 0)

    def _gather(c):
        idx = cidx_v.at[pl.ds(c * CH, CH)]
        return pltpu.async_copy(ctab_hbm.at[idx], rows_v.at[c % NB], gsem)

    def _scatter(c):
        return pltpu.async_copy(
            rows_v.at[c % NB], out_hbm.at[pl.ds(base + c * CH, CH)], ssem)

    gathers = [None] * NCH
    scatters = [None] * NCH
    for c in range(NB - 1):
        gathers[c] = _gather(c)
    for c in range(NCH):
        gathers[c].wait()
        nxt = c + NB - 1
        if nxt < NCH:
            if c - 1 >= 0:
                scatters[c - 1].wait()
            gathers[nxt] = _gather(nxt)
        scatters[c] = _scatter(c)
    for c in range(NCH - NB, NCH):
        scatters[c].wait()


@jax.jit
def _run(node_ids, depth_ids, ctab):
    k = functools.partial(
        pl.kernel,
        out_type=jax.ShapeDtypeStruct((N_TOK, D), jnp.float32),
        mesh=plsc.VectorSubcoreMesh(core_axis_name="c", subcore_axis_name="s"),
        scratch_types=[
            pltpu.VMEM((TOK_W,), jnp.int32),
            pltpu.VMEM((TOK_W,), jnp.int32),
            pltpu.VMEM((TOK_W,), jnp.int32),
            pltpu.VMEM((NB, CH, D), jnp.float32),
            pltpu.SemaphoreType.DMA,
            pltpu.SemaphoreType.DMA,
        ],
    )(_sc_body)
    return k(node_ids, depth_ids, ctab)


def kernel(node_type_ids, depth_ids, node_table, depth_table):
    b, t = node_type_ids.shape
    ctab = _build_ctable(node_table, depth_table)
    nid = node_type_ids.reshape(-1).astype(jnp.int32)
    did = depth_ids.reshape(-1).astype(jnp.int32)
    out = _run(nid, did, ctab)
    return out.reshape(b, t, D)
